# baseline (speedup 1.0000x reference)
import functools

import jax
import jax.numpy as jnp
from jax import lax
from jax.experimental import pallas as pl
from jax.experimental.pallas import tpu as pltpu
from jax.experimental.pallas import tpu_sc as plsc

N = 10000
E = 320000
D = 128
G = 64

NC = 2
NS = 16
NW = NC * NS
EP = 327680
EPT = EP // NW
C = 80
PH = 4
PE = EPT // PH
PCH = PE // C
NBUF = 3
NP = 10240
RPT = NP // NS

BN = 1000
NB = N // BN


def _sc_segment_sum(x, src, dst3d):
    mesh = plsc.VectorSubcoreMesh(core_axis_name="c", subcore_axis_name="s")

    @functools.partial(
        pl.kernel,
        mesh=mesh,
        out_type=jax.ShapeDtypeStruct((NC, NP, D), jnp.float32),
        scratch_types=[
            pltpu.VMEM((PE,), jnp.int32),
            pltpu.VMEM((PCH, C), jnp.int32),
            pltpu.VMEM((NBUF, C, D), jnp.float32),
            pltpu.VMEM_SHARED((NP, D), jnp.float32),
            pltpu.SemaphoreType.DMA,
            pltpu.SemaphoreType.DMA,
            pltpu.SemaphoreType.DMA,
        ],
    )
    def seg(x_hbm, src_hbm, dst_hbm, out_hbm,
            src_v, dst_v, rows_v, acc_sh, gs0, gs1, gs2):
        c = lax.axis_index("c")
        s = lax.axis_index("s")
        wid = s * NC + c
        gsems = [gs0, gs1, gs2]

        def zrow(i, carry):
            def zcol(l, carry2):
                rows_v[1, i, pl.ds(l * 16, 16)] = jnp.zeros((16,),
                                                            jnp.float32)
                return carry2
            return lax.fori_loop(0, D // 16, zcol, carry)
        lax.fori_loop(0, C, zrow, 0)

        def zcp(k, carry):
            pltpu.sync_copy(rows_v.at[1], acc_sh.at[pl.ds(s * RPT + k * C, C)])
            return carry
        lax.fori_loop(0, RPT // C, zcp, 0)

        plsc.subcore_barrier()

        def fire_g(j, b):
            pltpu.async_copy(
                x_hbm.at[src_v.at[pl.ds(j * C, C)]], rows_v.at[b], gsems[b])

        def wait_g(j, b):
            pltpu.make_async_copy(
                x_hbm.at[src_v.at[pl.ds(j * C, C)]], rows_v.at[b],
                gsems[b]).wait()

        def scat(j, b):
            pltpu.sync_copy(rows_v.at[b], acc_sh.at[dst_v.at[j]], add=True)

        def phase(p, carry):
            pltpu.sync_copy(src_hbm.at[pl.ds(wid * EPT + p * PE, PE)], src_v)
            pltpu.sync_copy(dst_hbm.at[wid, p], dst_v)

            fire_g(0, 0)

            def body(t, carry2):
                j0 = 2 * t
                fire_g(j0 + 1, 1)
                wait_g(j0, 0)
                scat(j0, 0)
                fire_g(j0 + 2, 0)
                wait_g(j0 + 1, 1)
                scat(j0 + 1, 1)
                return carry2
            lax.fori_loop(0, PCH // 2 - 1, body, 0)

            fire_g(PCH - 1, 1)
            wait_g(PCH - 2, 0)
            scat(PCH - 2, 0)
            wait_g(PCH - 1, 1)
            scat(PCH - 1, 1)
            return carry
        lax.fori_loop(0, PH, phase, 0)

        plsc.subcore_barrier()

        def wb(k, carry):
            base = s * RPT + k * C
            pltpu.sync_copy(acc_sh.at[pl.ds(base, C)],
                            out_hbm.at[c, pl.ds(base, C)])
            return carry
        lax.fori_loop(0, RPT // C, wb, 0)

    return seg(x, src, dst3d)


def _tc_mlp(x, parts, Wa, ba, Wb, bb):
    def body(x_ref, p_ref, wa, ba_r, wb, bb_r, o_ref):
        z = x_ref[...] + p_ref[0] + p_ref[1]
        h = jnp.maximum(
            jnp.dot(z, wa[...], preferred_element_type=jnp.float32)
            + ba_r[...], 0.0)
        h = jnp.dot(h, wb[...], preferred_element_type=jnp.float32) + bb_r[...]
        o_ref[...] = jnp.maximum(h, 0.0)

    return pl.pallas_call(
        body,
        grid=(NB,),
        in_specs=[
            pl.BlockSpec((BN, D), lambda i: (i, 0)),
            pl.BlockSpec((NC, BN, D), lambda i: (0, i, 0)),
            pl.BlockSpec((D, D), lambda i: (0, 0)),
            pl.BlockSpec((1, D), lambda i: (0, 0)),
            pl.BlockSpec((D, D), lambda i: (0, 0)),
            pl.BlockSpec((1, D), lambda i: (0, 0)),
        ],
        out_specs=pl.BlockSpec((BN, D), lambda i: (i, 0)),
        out_shape=jax.ShapeDtypeStruct((N, D), jnp.float32),
    )(x, parts, Wa, ba.reshape(1, D), Wb, bb.reshape(1, D))


def _tc_mlp_pool(h1, parts, Wa, ba, Wb, bb, batch3, fc_w, fc_b):
    def body(h_ref, p_ref, wa, ba_r, wb, bb_r, b_ref, fw, fb,
             o_ref, acc, cnt):
        i = pl.program_id(0)

        @pl.when(i == 0)
        def _():
            acc[...] = jnp.zeros_like(acc)
            cnt[...] = jnp.zeros_like(cnt)

        z = h_ref[...] + p_ref[0] + p_ref[1]
        h = jnp.maximum(
            jnp.dot(z, wa[...], preferred_element_type=jnp.float32)
            + ba_r[...], 0.0)
        h = jnp.maximum(
            jnp.dot(h, wb[...], preferred_element_type=jnp.float32)
            + bb_r[...], 0.0)

        b = b_ref[0, 0, :]
        ohT = (lax.broadcasted_iota(jnp.int32, (G, BN), 0)
               == b[None, :]).astype(jnp.float32)
        acc[...] += jnp.dot(ohT, h, preferred_element_type=jnp.float32)
        cnt[...] += jnp.sum(ohT, axis=1, keepdims=True)

        @pl.when(i == NB - 1)
        def _():
            pooled = acc[...] / jnp.maximum(cnt[...], 1.0)
            o_ref[...] = (jnp.dot(pooled, fw[...],
                                  preferred_element_type=jnp.float32)
                          + fb[...])

    return pl.pallas_call(
        body,
        grid=(NB,),
        in_specs=[
            pl.BlockSpec((BN, D), lambda i: (i, 0)),
            pl.BlockSpec((NC, BN, D), lambda i: (0, i, 0)),
            pl.BlockSpec((D, D), lambda i: (0, 0)),
            pl.BlockSpec((1, D), lambda i: (0, 0)),
            pl.BlockSpec((D, D), lambda i: (0, 0)),
            pl.BlockSpec((1, D), lambda i: (0, 0)),
            pl.BlockSpec((1, 1, BN), lambda i: (i, 0, 0)),
            pl.BlockSpec((D, 1), lambda i: (0, 0)),
            pl.BlockSpec((1, 1), lambda i: (0, 0)),
        ],
        out_specs=pl.BlockSpec((G, 1), lambda i: (0, 0)),
        out_shape=jax.ShapeDtypeStruct((G, 1), jnp.float32),
        scratch_shapes=[
            pltpu.VMEM((G, D), jnp.float32),
            pltpu.VMEM((G, 1), jnp.float32),
        ],
    )(h1, parts, Wa, ba.reshape(1, D), Wb, bb.reshape(1, D),
      batch3, fc_w, fc_b.reshape(1, 1))


def kernel(x, edge_index, batch, W1a, b1a, W1b, b1b, W2a, b2a, W2b, b2b,
           fc_w, fc_b):
    npad = EP - E
    src = jnp.concatenate([edge_index[0], jnp.zeros((npad,), jnp.int32)])
    pad_dst = N + (jnp.arange(npad, dtype=jnp.int32) % (NP - N))
    dst = jnp.concatenate([edge_index[1], pad_dst])
    dst4d = dst.reshape(NW, PH, PCH, C)
    batch3 = batch.reshape(NB, 1, BN)

    parts1 = _sc_segment_sum(x, src, dst4d)
    h1 = _tc_mlp(x, parts1, W1a, b1a, W1b, b1b)
    parts2 = _sc_segment_sum(h1, src, dst4d)
    out = _tc_mlp_pool(h1, parts2, W2a, b2a, W2b, b2b, batch3, fc_w, fc_b)
    return out.reshape(G)


# speedup vs baseline: 1.8741x; 1.8741x over previous
import functools

import jax
import jax.numpy as jnp
from jax import lax
from jax.experimental import pallas as pl
from jax.experimental.pallas import tpu as pltpu
from jax.experimental.pallas import tpu_sc as plsc

N = 10000
E = 320000
D = 128
G = 64

NC = 2
NS = 16
NW = NC * NS
C = 96
NCH = 105
EPT = NCH * C
EP = EPT * NW
NP = 10240
RPT = NP // NS
WC = 80

BN = 1000
NB = N // BN


def _sc_segment_sum(x, src, dst3d):
    mesh = plsc.VectorSubcoreMesh(core_axis_name="c", subcore_axis_name="s")

    @functools.partial(
        pl.kernel,
        mesh=mesh,
        out_type=jax.ShapeDtypeStruct((NC, NP, D), jnp.float32),
        scratch_types=[
            pltpu.VMEM((EPT,), jnp.int32),
            pltpu.VMEM((NCH, C), jnp.int32),
            pltpu.VMEM((2, C, D), jnp.float32),
            pltpu.VMEM_SHARED((NP, D), jnp.float32),
            pltpu.SemaphoreType.DMA,
            pltpu.SemaphoreType.DMA,
        ],
    )
    def seg(x_hbm, src_hbm, dst_hbm, out_hbm,
            src_v, dst_v, rows_v, acc_sh, sem_a, sem_b):
        c = lax.axis_index("c")
        s = lax.axis_index("s")
        wid = s * NC + c

        idx_cp_a = pltpu.async_copy(
            src_hbm.at[pl.ds(wid * EPT, EPT)], src_v, sem_a)
        idx_cp_b = pltpu.async_copy(dst_hbm.at[wid], dst_v, sem_b)

        def zrow(i, carry):
            def zcol(l, carry2):
                rows_v[1, i, pl.ds(l * 16, 16)] = jnp.zeros((16,),
                                                            jnp.float32)
                return carry2
            return lax.fori_loop(0, D // 16, zcol, carry)
        lax.fori_loop(0, WC, zrow, 0)

        def zcp(k, carry):
            pltpu.sync_copy(rows_v.at[1, pl.ds(0, WC)],
                            acc_sh.at[pl.ds(s * RPT + k * WC, WC)])
            return carry
        lax.fori_loop(0, RPT // WC, zcp, 0)

        idx_cp_a.wait()
        idx_cp_b.wait()

        plsc.subcore_barrier()

        def gath(j, buf, sem):
            return pltpu.async_copy(
                x_hbm.at[src_v.at[pl.ds(j * C, C)]], rows_v.at[buf], sem)

        gath(0, 0, sem_a)

        def body(t, carry):
            j0 = 2 * t
            gath(j0 + 1, 1, sem_b)
            pltpu.make_async_copy(
                x_hbm.at[src_v.at[pl.ds(j0 * C, C)]], rows_v.at[0],
                sem_a).wait()
            pltpu.sync_copy(rows_v.at[0], acc_sh.at[dst_v.at[j0]], add=True)
            gath(j0 + 2, 0, sem_a)
            pltpu.make_async_copy(
                x_hbm.at[src_v.at[pl.ds((j0 + 1) * C, C)]], rows_v.at[1],
                sem_b).wait()
            pltpu.sync_copy(rows_v.at[1], acc_sh.at[dst_v.at[j0 + 1]],
                            add=True)
            return carry
        lax.fori_loop(0, (NCH - 1) // 2, body, 0)

        jl = NCH - 1
        pltpu.make_async_copy(
            x_hbm.at[src_v.at[pl.ds(jl * C, C)]], rows_v.at[0], sem_a).wait()
        pltpu.sync_copy(rows_v.at[0], acc_sh.at[dst_v.at[jl]], add=True)

        plsc.subcore_barrier()

        def wb(k, carry):
            base = s * RPT + k * WC
            pltpu.sync_copy(acc_sh.at[pl.ds(base, WC)],
                            out_hbm.at[c, pl.ds(base, WC)])
            return carry
        lax.fori_loop(0, RPT // WC, wb, 0)

    return seg(x, src, dst3d)


def _tc_mlp(x, parts, Wa, ba, Wb, bb):
    def body(x_ref, p_ref, wa, ba_r, wb, bb_r, o_ref):
        z = x_ref[...] + p_ref[0] + p_ref[1]
        h = jnp.maximum(
            jnp.dot(z, wa[...], preferred_element_type=jnp.float32)
            + ba_r[...], 0.0)
        h = jnp.dot(h, wb[...], preferred_element_type=jnp.float32) + bb_r[...]
        o_ref[...] = jnp.maximum(h, 0.0)

    return pl.pallas_call(
        body,
        grid=(NB,),
        in_specs=[
            pl.BlockSpec((BN, D), lambda i: (i, 0)),
            pl.BlockSpec((NC, BN, D), lambda i: (0, i, 0)),
            pl.BlockSpec((D, D), lambda i: (0, 0)),
            pl.BlockSpec((1, D), lambda i: (0, 0)),
            pl.BlockSpec((D, D), lambda i: (0, 0)),
            pl.BlockSpec((1, D), lambda i: (0, 0)),
        ],
        out_specs=pl.BlockSpec((BN, D), lambda i: (i, 0)),
        out_shape=jax.ShapeDtypeStruct((N, D), jnp.float32),
    )(x, parts, Wa, ba.reshape(1, D), Wb, bb.reshape(1, D))


def _tc_mlp_pool(h1, parts, Wa, ba, Wb, bb, batch3, fc_w, fc_b):
    def body(h_ref, p_ref, wa, ba_r, wb, bb_r, b_ref, fw, fb,
             o_ref, acc, cnt):
        i = pl.program_id(0)

        @pl.when(i == 0)
        def _():
            acc[...] = jnp.zeros_like(acc)
            cnt[...] = jnp.zeros_like(cnt)

        z = h_ref[...] + p_ref[0] + p_ref[1]
        h = jnp.maximum(
            jnp.dot(z, wa[...], preferred_element_type=jnp.float32)
            + ba_r[...], 0.0)
        h = jnp.maximum(
            jnp.dot(h, wb[...], preferred_element_type=jnp.float32)
            + bb_r[...], 0.0)

        b = b_ref[0, 0, :]
        ohT = (lax.broadcasted_iota(jnp.int32, (G, BN), 0)
               == b[None, :]).astype(jnp.float32)
        acc[...] += jnp.dot(ohT, h, preferred_element_type=jnp.float32)
        cnt[...] += jnp.sum(ohT, axis=1, keepdims=True)

        @pl.when(i == NB - 1)
        def _():
            pooled = acc[...] / jnp.maximum(cnt[...], 1.0)
            o_ref[...] = (jnp.dot(pooled, fw[...],
                                  preferred_element_type=jnp.float32)
                          + fb[...])

    return pl.pallas_call(
        body,
        grid=(NB,),
        in_specs=[
            pl.BlockSpec((BN, D), lambda i: (i, 0)),
            pl.BlockSpec((NC, BN, D), lambda i: (0, i, 0)),
            pl.BlockSpec((D, D), lambda i: (0, 0)),
            pl.BlockSpec((1, D), lambda i: (0, 0)),
            pl.BlockSpec((D, D), lambda i: (0, 0)),
            pl.BlockSpec((1, D), lambda i: (0, 0)),
            pl.BlockSpec((1, 1, BN), lambda i: (i, 0, 0)),
            pl.BlockSpec((D, 1), lambda i: (0, 0)),
            pl.BlockSpec((1, 1), lambda i: (0, 0)),
        ],
        out_specs=pl.BlockSpec((G, 1), lambda i: (0, 0)),
        out_shape=jax.ShapeDtypeStruct((G, 1), jnp.float32),
        scratch_shapes=[
            pltpu.VMEM((G, D), jnp.float32),
            pltpu.VMEM((G, 1), jnp.float32),
        ],
    )(h1, parts, Wa, ba.reshape(1, D), Wb, bb.reshape(1, D),
      batch3, fc_w, fc_b.reshape(1, 1))


def kernel(x, edge_index, batch, W1a, b1a, W1b, b1b, W2a, b2a, W2b, b2b,
           fc_w, fc_b):
    npad = EP - E
    src = jnp.concatenate([edge_index[0], jnp.zeros((npad,), jnp.int32)])
    pad_dst = N + (jnp.arange(npad, dtype=jnp.int32) % (NP - N))
    dst3d = jnp.concatenate([edge_index[1], pad_dst]).reshape(NW, NCH, C)
    batch3 = batch.reshape(NB, 1, BN)

    parts1 = _sc_segment_sum(x, src, dst3d)
    h1 = _tc_mlp(x, parts1, W1a, b1a, W1b, b1b)
    parts2 = _sc_segment_sum(h1, src, dst3d)
    out = _tc_mlp_pool(h1, parts2, W2a, b2a, W2b, b2b, batch3, fc_w, fc_b)
    return out.reshape(G)


# speedup vs baseline: 3.3488x; 1.7869x over previous
import functools

import jax
import jax.numpy as jnp
from jax import lax
from jax.experimental import pallas as pl
from jax.experimental.pallas import tpu as pltpu
from jax.experimental.pallas import tpu_sc as plsc

N = 10000
E = 320000
D = 128
G = 64

NC = 2
NS = 16
NW = NC * NS
EPT = E // NW
C = 80
NCH = EPT // C
NP = 10240
RPT = NP // NS

BN = 1000
NB = N // BN


def _sc_segment_sum(x, src, dst3d):
    mesh = plsc.VectorSubcoreMesh(core_axis_name="c", subcore_axis_name="s")

    @functools.partial(
        pl.kernel,
        mesh=mesh,
        out_type=jax.ShapeDtypeStruct((NC, NP, D), jnp.float32),
        scratch_types=[
            pltpu.VMEM((EPT,), jnp.int32),
            pltpu.VMEM((NCH, C), jnp.int32),
            pltpu.VMEM((2, C, D), jnp.float32),
            pltpu.VMEM_SHARED((NP, D), jnp.float32),
            pltpu.SemaphoreType.DMA,
            pltpu.SemaphoreType.DMA,
        ],
    )
    def seg(x_hbm, src_hbm, dst_hbm, out_hbm,
            src_v, dst_v, rows_v, acc_sh, sem_a, sem_b):
        c = lax.axis_index("c")
        s = lax.axis_index("s")
        wid = s * NC + c

        idx_cp_a = pltpu.async_copy(
            src_hbm.at[pl.ds(wid * EPT, EPT)], src_v, sem_a)
        idx_cp_b = pltpu.async_copy(dst_hbm.at[wid], dst_v, sem_b)

        def zrow(i, carry):
            def zcol(l, carry2):
                rows_v[1, i, pl.ds(l * 16, 16)] = jnp.zeros((16,),
                                                            jnp.float32)
                return carry2
            return lax.fori_loop(0, D // 16, zcol, carry)
        lax.fori_loop(0, C, zrow, 0)

        def zcp(k, carry):
            pltpu.sync_copy(rows_v.at[1], acc_sh.at[pl.ds(s * RPT + k * C, C)])
            return carry
        lax.fori_loop(0, RPT // C, zcp, 0)

        idx_cp_a.wait()
        idx_cp_b.wait()

        def gath(j, buf, sem):
            return pltpu.async_copy(
                x_hbm.at[src_v.at[pl.ds(j * C, C)]], rows_v.at[buf], sem)

        gath(0, 0, sem_a)
        gath(1, 1, sem_b)

        plsc.subcore_barrier()

        def body(t, carry):
            j0 = 2 * t
            pltpu.make_async_copy(
                x_hbm.at[src_v.at[pl.ds(j0 * C, C)]], rows_v.at[0],
                sem_a).wait()
            pltpu.sync_copy(rows_v.at[0], acc_sh.at[dst_v.at[j0]], add=True)
            gath(j0 + 2, 0, sem_a)
            pltpu.make_async_copy(
                x_hbm.at[src_v.at[pl.ds((j0 + 1) * C, C)]], rows_v.at[1],
                sem_b).wait()
            pltpu.sync_copy(rows_v.at[1], acc_sh.at[dst_v.at[j0 + 1]],
                            add=True)
            gath(j0 + 3, 1, sem_b)
            return carry
        lax.fori_loop(0, (NCH - 3) // 2, body, 0)

        j2, j1, j0e = NCH - 3, NCH - 2, NCH - 1
        pltpu.make_async_copy(
            x_hbm.at[src_v.at[pl.ds(j2 * C, C)]], rows_v.at[0], sem_a).wait()
        pltpu.sync_copy(rows_v.at[0], acc_sh.at[dst_v.at[j2]], add=True)
        gath(j0e, 0, sem_a)
        pltpu.make_async_copy(
            x_hbm.at[src_v.at[pl.ds(j1 * C, C)]], rows_v.at[1], sem_b).wait()
        pltpu.sync_copy(rows_v.at[1], acc_sh.at[dst_v.at[j1]], add=True)
        pltpu.make_async_copy(
            x_hbm.at[src_v.at[pl.ds(j0e * C, C)]], rows_v.at[0], sem_a).wait()
        pltpu.sync_copy(rows_v.at[0], acc_sh.at[dst_v.at[j0e]], add=True)

        plsc.subcore_barrier()

        def wb(k, carry):
            base = s * RPT + k * C
            pltpu.sync_copy(acc_sh.at[pl.ds(base, C)],
                            out_hbm.at[c, pl.ds(base, C)])
            return carry
        lax.fori_loop(0, RPT // C, wb, 0)

    return seg(x, src, dst3d)


def _tc_mlp(x, parts, Wa, ba, Wb, bb):
    def body(x_ref, p_ref, wa, ba_r, wb, bb_r, o_ref):
        z = x_ref[...] + p_ref[0] + p_ref[1]
        h = jnp.maximum(
            jnp.dot(z, wa[...], preferred_element_type=jnp.float32)
            + ba_r[...], 0.0)
        h = jnp.dot(h, wb[...], preferred_element_type=jnp.float32) + bb_r[...]
        o_ref[...] = jnp.maximum(h, 0.0)

    return pl.pallas_call(
        body,
        grid=(NB,),
        in_specs=[
            pl.BlockSpec((BN, D), lambda i: (i, 0)),
            pl.BlockSpec((NC, BN, D), lambda i: (0, i, 0)),
            pl.BlockSpec((D, D), lambda i: (0, 0)),
            pl.BlockSpec((1, D), lambda i: (0, 0)),
            pl.BlockSpec((D, D), lambda i: (0, 0)),
            pl.BlockSpec((1, D), lambda i: (0, 0)),
        ],
        out_specs=pl.BlockSpec((BN, D), lambda i: (i, 0)),
        out_shape=jax.ShapeDtypeStruct((N, D), jnp.float32),
    )(x, parts, Wa, ba.reshape(1, D), Wb, bb.reshape(1, D))


def _tc_mlp_pool(h1, parts, Wa, ba, Wb, bb, batch3, fc_w, fc_b):
    def body(h_ref, p_ref, wa, ba_r, wb, bb_r, b_ref, fw, fb,
             o_ref, acc, cnt):
        i = pl.program_id(0)

        @pl.when(i == 0)
        def _():
            acc[...] = jnp.zeros_like(acc)
            cnt[...] = jnp.zeros_like(cnt)

        z = h_ref[...] + p_ref[0] + p_ref[1]
        h = jnp.maximum(
            jnp.dot(z, wa[...], preferred_element_type=jnp.float32)
            + ba_r[...], 0.0)
        h = jnp.maximum(
            jnp.dot(h, wb[...], preferred_element_type=jnp.float32)
            + bb_r[...], 0.0)

        b = b_ref[0, 0, :]
        ohT = (lax.broadcasted_iota(jnp.int32, (G, BN), 0)
               == b[None, :]).astype(jnp.float32)
        acc[...] += jnp.dot(ohT, h, preferred_element_type=jnp.float32)
        cnt[...] += jnp.sum(ohT, axis=1, keepdims=True)

        @pl.when(i == NB - 1)
        def _():
            pooled = acc[...] / jnp.maximum(cnt[...], 1.0)
            o_ref[...] = (jnp.dot(pooled, fw[...],
                                  preferred_element_type=jnp.float32)
                          + fb[...])

    return pl.pallas_call(
        body,
        grid=(NB,),
        in_specs=[
            pl.BlockSpec((BN, D), lambda i: (i, 0)),
            pl.BlockSpec((NC, BN, D), lambda i: (0, i, 0)),
            pl.BlockSpec((D, D), lambda i: (0, 0)),
            pl.BlockSpec((1, D), lambda i: (0, 0)),
            pl.BlockSpec((D, D), lambda i: (0, 0)),
            pl.BlockSpec((1, D), lambda i: (0, 0)),
            pl.BlockSpec((1, 1, BN), lambda i: (i, 0, 0)),
            pl.BlockSpec((D, 1), lambda i: (0, 0)),
            pl.BlockSpec((1, 1), lambda i: (0, 0)),
        ],
        out_specs=pl.BlockSpec((G, 1), lambda i: (0, 0)),
        out_shape=jax.ShapeDtypeStruct((G, 1), jnp.float32),
        scratch_shapes=[
            pltpu.VMEM((G, D), jnp.float32),
            pltpu.VMEM((G, 1), jnp.float32),
        ],
    )(h1, parts, Wa, ba.reshape(1, D), Wb, bb.reshape(1, D),
      batch3, fc_w, fc_b.reshape(1, 1))


def kernel(x, edge_index, batch, W1a, b1a, W1b, b1b, W2a, b2a, W2b, b2b,
           fc_w, fc_b):
    src = edge_index[0]
    dst3d = edge_index[1].reshape(NW, NCH, C)
    batch3 = batch.reshape(NB, 1, BN)

    parts1 = _sc_segment_sum(x, src, dst3d)
    h1 = _tc_mlp(x, parts1, W1a, b1a, W1b, b1b)
    parts2 = _sc_segment_sum(h1, src, dst3d)
    out = _tc_mlp_pool(h1, parts2, W2a, b2a, W2b, b2b, batch3, fc_w, fc_b)
    return out.reshape(G)
